# Initial kernel scaffold; baseline (speedup 1.0000x reference)
import jax
import jax.numpy as jnp
from jax.experimental import pallas as pl


def kernel(pillar_features, coords):
    raise NotImplementedError("write your pallas kernel here")


# speedup vs baseline: 1.8860x; 1.8860x over previous
import functools

import jax
import jax.numpy as jnp
from jax import lax
from jax.experimental import pallas as pl
from jax.experimental.pallas import tpu as pltpu
from jax.experimental.pallas import tpu_sc as plsc

NX = 512
NY = 512
C = 128
P = 60000
S = NX * NY

CH = 128
NCHUNK = (P + CH - 1) // CH
LAST_BASE = P - CH
T = 2048
L = 16


def _chunk_base(j):
    return jnp.minimum(j * CH, LAST_BASE)


def _idx_mask_body(coords_hbm,
                   idx_hbm, mask_hbm,
                   coords_v, idxc_v, idx_full_v, win_v, sem):
    w = lax.axis_index("s")

    lanes = lax.iota(jnp.int32, L)
    lanes4 = lanes * 4

    nk = (NCHUNK + 15) // 16

    def chunk_step(k, _):
        j = w + k * 16

        @pl.when(j < NCHUNK)
        def _():
            base = _chunk_base(j)
            pltpu.sync_copy(coords_hbm.at[pl.ds(base * 4, CH * 4)], coords_v)
            for i in range(CH // L):
                y = plsc.load_gather(coords_v, [lanes4 + (i * L * 4 + 2)])
                x = plsc.load_gather(coords_v, [lanes4 + (i * L * 4 + 3)])
                idxc_v[0, pl.ds(i * L, L)] = y * NX + x
            pltpu.sync_copy(idxc_v.at[0], idx_hbm.at[pl.ds(base, CH)])

        return 0

    lax.fori_loop(0, nk, chunk_step, 0)

    plsc.subcore_barrier()

    win = S // 16
    lo = w * win

    def zero_step(i, _):
        win_v[pl.ds(i * L, L)] = jnp.zeros((L,), jnp.int32)
        return 0
    lax.fori_loop(0, win // L, zero_step, 0)

    pltpu.sync_copy(idx_hbm, idx_full_v)

    ones16 = jnp.full((L,), 1, jnp.int32)

    def scan_step(i, _):
        v = idx_full_v[pl.ds(i * L, L)]
        local = v - lo
        m = (local >= 0) & (local < win)
        plsc.store_scatter(win_v, [jnp.where(m, local, 0)], ones16, mask=m)
        return 0
    lax.fori_loop(0, P // L, scan_step, 0)

    pltpu.sync_copy(win_v, mask_hbm.at[pl.ds(lo, win)])


@functools.partial(jax.jit)
def _idx_mask_call(coords):
    mesh = plsc.VectorSubcoreMesh(core_axis_name="c", subcore_axis_name="s",
                                  num_cores=1)
    f = pl.kernel(
        _idx_mask_body,
        out_type=(jax.ShapeDtypeStruct((P,), jnp.int32),
                  jax.ShapeDtypeStruct((S,), jnp.int32)),
        mesh=mesh,
        scratch_types=[
            pltpu.VMEM((CH * 4,), jnp.int32),
            pltpu.VMEM((1, CH), jnp.int32),
            pltpu.VMEM((P,), jnp.int32),
            pltpu.VMEM((S // 16,), jnp.int32),
            pltpu.SemaphoreType.DMA,
        ],
        compiler_params=pltpu.CompilerParams(needs_layout_passes=False,
                                             use_tc_tiling_on_sc=False),
    )
    return f(coords)


def _scatter_body(feat_hbm, idx_hbm, inter_hbm, feat_v, idxc_v, sem):
    wid = lax.axis_index("s") * 2 + lax.axis_index("c")

    nk = (NCHUNK + 31) // 32

    def chunk_step(k, _):
        j = wid + k * 32

        @pl.when(j < NCHUNK)
        def _():
            base = _chunk_base(j)
            pltpu.sync_copy(idx_hbm.at[pl.ds(base, CH)], idxc_v.at[0])
            pltpu.sync_copy(feat_hbm.at[pl.ds(base, CH), :], feat_v)
            pltpu.async_copy(feat_v, inter_hbm.at[idxc_v.at[0]], sem).wait()

        return 0

    lax.fori_loop(0, nk, chunk_step, 0)


@functools.partial(jax.jit)
def _scatter_call(feat, idx):
    mesh = plsc.VectorSubcoreMesh(core_axis_name="c", subcore_axis_name="s")
    f = pl.kernel(
        _scatter_body,
        out_type=jax.ShapeDtypeStruct((S, C), jnp.float32),
        mesh=mesh,
        scratch_types=[
            pltpu.VMEM((CH, C), jnp.float32),
            pltpu.VMEM((1, CH), jnp.int32),
            pltpu.SemaphoreType.DMA,
        ],
        compiler_params=pltpu.CompilerParams(needs_layout_passes=False),
    )
    return f(feat, idx)


def _transpose_body(inter_ref, mask_ref, out_ref):
    x = inter_ref[...]
    xt = x.T
    m = mask_ref[0, 0, :]
    out_ref[...] = jnp.where((m != 0)[None, :], xt, 0.0)


@functools.partial(jax.jit)
def _transpose_call(inter, mask3d):
    grid = (S // T,)
    return pl.pallas_call(
        _transpose_body,
        grid=grid,
        in_specs=[
            pl.BlockSpec((T, C), lambda g: (g, 0)),
            pl.BlockSpec((1, 1, T), lambda g: (g, 0, 0)),
        ],
        out_specs=pl.BlockSpec((C, T), lambda g: (0, g)),
        out_shape=jax.ShapeDtypeStruct((C, S), jnp.float32),
    )(inter, mask3d)


def kernel(pillar_features, coords):
    idx, mask = _idx_mask_call(coords.reshape(-1))
    inter = _scatter_call(pillar_features, idx)
    mask3d = mask.reshape(S // T, 1, T)
    out2 = _transpose_call(inter, mask3d)
    return out2.reshape(1, C, NY, NX)


# speedup vs baseline: 2.7677x; 1.4675x over previous
import functools

import jax
import jax.numpy as jnp
from jax import lax
from jax.experimental import pallas as pl
from jax.experimental.pallas import tpu as pltpu
from jax.experimental.pallas import tpu_sc as plsc

NX = 512
NY = 512
C = 128
P = 60000
S = NX * NY

CH = 128
NCHUNK = (P + CH - 1) // CH
LAST_BASE = P - CH
T = 4096
L = 16


def _chunk_base(j):
    return jnp.minimum(j * CH, LAST_BASE)


def _idx_mask_body(coords_hbm,
                   idx_hbm, mask_hbm,
                   coords_v, idxc_v, idx_full_v, win_v, sem):
    w = lax.axis_index("s")

    lanes = lax.iota(jnp.int32, L)
    lanes4 = lanes * 4

    nk = (NCHUNK + 15) // 16

    def chunk_step(k, _):
        j = w + k * 16

        @pl.when(j < NCHUNK)
        def _():
            base = _chunk_base(j)
            pltpu.sync_copy(coords_hbm.at[pl.ds(base * 4, CH * 4)], coords_v)
            for i in range(CH // L):
                y = plsc.load_gather(coords_v, [lanes4 + (i * L * 4 + 2)])
                x = plsc.load_gather(coords_v, [lanes4 + (i * L * 4 + 3)])
                idxc_v[0, pl.ds(i * L, L)] = y * NX + x
            pltpu.sync_copy(idxc_v.at[0], idx_hbm.at[pl.ds(base, CH)])

        return 0

    lax.fori_loop(0, nk, chunk_step, 0)

    plsc.subcore_barrier()

    win = S // 16
    lo = w * win

    def zero_step(i, _):
        win_v[pl.ds(i * L, L)] = jnp.zeros((L,), jnp.int32)
        return 0
    lax.fori_loop(0, win // L, zero_step, 0)

    pltpu.sync_copy(idx_hbm, idx_full_v)

    ones16 = jnp.full((L,), 1, jnp.int32)

    def scan_step(i, _):
        v = idx_full_v[pl.ds(i * L, L)]
        local = v - lo
        m = (local >= 0) & (local < win)
        plsc.store_scatter(win_v, [jnp.where(m, local, 0)], ones16, mask=m)
        return 0
    lax.fori_loop(0, P // L, scan_step, 0)

    pltpu.sync_copy(win_v, mask_hbm.at[pl.ds(lo, win)])


@functools.partial(jax.jit)
def _idx_mask_call(coords):
    mesh = plsc.VectorSubcoreMesh(core_axis_name="c", subcore_axis_name="s",
                                  num_cores=1)
    f = pl.kernel(
        _idx_mask_body,
        out_type=(jax.ShapeDtypeStruct((P,), jnp.int32),
                  jax.ShapeDtypeStruct((S,), jnp.int32)),
        mesh=mesh,
        scratch_types=[
            pltpu.VMEM((CH * 4,), jnp.int32),
            pltpu.VMEM((1, CH), jnp.int32),
            pltpu.VMEM((P,), jnp.int32),
            pltpu.VMEM((S // 16,), jnp.int32),
            pltpu.SemaphoreType.DMA,
        ],
        compiler_params=pltpu.CompilerParams(needs_layout_passes=False,
                                             use_tc_tiling_on_sc=False),
    )
    return f(coords)


def _scatter_body(feat_hbm, idx_hbm, inter_hbm, feat_v, idxc_v, sem):
    wid = lax.axis_index("s") * 2 + lax.axis_index("c")

    nk = (NCHUNK + 31) // 32

    def chunk_step(k, _):
        j = wid + k * 32

        @pl.when(j < NCHUNK)
        def _():
            base = _chunk_base(j)
            pltpu.sync_copy(idx_hbm.at[pl.ds(base, CH)], idxc_v.at[0])
            pltpu.sync_copy(feat_hbm.at[pl.ds(base, CH), :], feat_v)
            pltpu.async_copy(feat_v, inter_hbm.at[idxc_v.at[0]], sem).wait()

        return 0

    lax.fori_loop(0, nk, chunk_step, 0)


@functools.partial(jax.jit)
def _scatter_call(feat, idx):
    mesh = plsc.VectorSubcoreMesh(core_axis_name="c", subcore_axis_name="s")
    f = pl.kernel(
        _scatter_body,
        out_type=jax.ShapeDtypeStruct((S, C), jnp.float32),
        mesh=mesh,
        scratch_types=[
            pltpu.VMEM((CH, C), jnp.float32),
            pltpu.VMEM((1, CH), jnp.int32),
            pltpu.SemaphoreType.DMA,
        ],
        compiler_params=pltpu.CompilerParams(needs_layout_passes=False),
    )
    return f(feat, idx)


def _transpose_body(inter_ref, mask_ref, out_ref):
    x = inter_ref[...]
    xt = x.T
    m = mask_ref[0, 0, :]
    o = jnp.where((m != 0)[None, :], xt, 0.0)
    out_ref[...] = o.reshape(C, T // NX, NX)


@functools.partial(jax.jit)
def _transpose_call(inter, mask3d):
    grid = (S // T,)
    yb = T // NX
    return pl.pallas_call(
        _transpose_body,
        grid=grid,
        in_specs=[
            pl.BlockSpec((T, C), lambda g: (g, 0)),
            pl.BlockSpec((1, 1, T), lambda g: (g, 0, 0)),
        ],
        out_specs=pl.BlockSpec((C, yb, NX), lambda g: (0, g, 0)),
        out_shape=jax.ShapeDtypeStruct((C, NY, NX), jnp.float32),
    )(inter, mask3d)


def kernel(pillar_features, coords):
    idx, mask = _idx_mask_call(coords.reshape(-1))
    inter = _scatter_call(pillar_features, idx)
    mask3d = mask.reshape(S // T, 1, T)
    out3 = _transpose_call(inter, mask3d)
    return out3.reshape(1, C, NY, NX)


# speedup vs baseline: 3.1246x; 1.1289x over previous
import functools

import jax
import jax.numpy as jnp
from jax import lax
from jax.experimental import pallas as pl
from jax.experimental.pallas import tpu as pltpu
from jax.experimental.pallas import tpu_sc as plsc

NX = 512
NY = 512
C = 128
P = 60000
S = NX * NY

CH = 128
NCHUNK = (P + CH - 1) // CH
LAST_BASE = P - CH
T = 4096
L = 16


def _chunk_base(j):
    return jnp.minimum(j * CH, LAST_BASE)


FULL = 3752
LAST = P - 15 * FULL


def _idx_mask_body(coords_hbm,
                   idx_hbm, mask_hbm,
                   coords_v, idxst_v, idx_full_v, win_v, sem):
    w = lax.axis_index("s")

    lanes4 = lax.iota(jnp.int32, L) * 4
    start = w * FULL

    @pl.when(w < 15)
    def _():
        pltpu.sync_copy(coords_hbm.at[pl.ds(start * 4, FULL * 4)],
                        coords_v.at[pl.ds(0, FULL * 4)])

    @pl.when(w == 15)
    def _():
        pltpu.sync_copy(coords_hbm.at[pl.ds(start * 4, LAST * 4)],
                        coords_v.at[pl.ds(0, LAST * 4)])

    niter = jnp.where(w < 15, (FULL + L - 1) // L, (LAST + L - 1) // L)

    def dec_step(i, _):
        base4 = i * (L * 4)
        y = plsc.load_gather(coords_v, [lanes4 + base4 + 2])
        x = plsc.load_gather(coords_v, [lanes4 + base4 + 3])
        idxst_v[pl.ds(i * L, L)] = y * NX + x
        return 0

    lax.fori_loop(0, niter, dec_step, 0)

    @pl.when(w < 15)
    def _():
        pltpu.sync_copy(idxst_v.at[pl.ds(0, FULL)],
                        idx_hbm.at[pl.ds(start, FULL)])

    @pl.when(w == 15)
    def _():
        pltpu.sync_copy(idxst_v.at[pl.ds(0, LAST)],
                        idx_hbm.at[pl.ds(start, LAST)])

    plsc.subcore_barrier()

    win = S // 16
    lo = w * win

    def zero_step(i, _):
        win_v[pl.ds(i * L, L)] = jnp.zeros((L,), jnp.int32)
        return 0
    lax.fori_loop(0, win // L, zero_step, 0)

    pltpu.sync_copy(idx_hbm, idx_full_v)

    ones16 = jnp.full((L,), 1, jnp.int32)

    def scan_step(i, _):
        v = idx_full_v[pl.ds(i * L, L)]
        local = v - lo
        m = (local >= 0) & (local < win)
        plsc.store_scatter(win_v, [jnp.where(m, local, 0)], ones16, mask=m)
        return 0
    lax.fori_loop(0, P // L, scan_step, 0)

    pltpu.sync_copy(win_v, mask_hbm.at[pl.ds(lo, win)])


@functools.partial(jax.jit)
def _idx_mask_call(coords):
    mesh = plsc.VectorSubcoreMesh(core_axis_name="c", subcore_axis_name="s",
                                  num_cores=1)
    f = pl.kernel(
        _idx_mask_body,
        out_type=(jax.ShapeDtypeStruct((P,), jnp.int32),
                  jax.ShapeDtypeStruct((S,), jnp.int32)),
        mesh=mesh,
        scratch_types=[
            pltpu.VMEM((FULL * 4 + 2 * L * 4,), jnp.int32),
            pltpu.VMEM((FULL + L,), jnp.int32),
            pltpu.VMEM((P,), jnp.int32),
            pltpu.VMEM((S // 16,), jnp.int32),
            pltpu.SemaphoreType.DMA,
        ],
        compiler_params=pltpu.CompilerParams(needs_layout_passes=False,
                                             use_tc_tiling_on_sc=False),
    )
    return f(coords)


def _scatter_body(feat_hbm, idx_hbm, inter_hbm,
                  feat_v0, feat_v1, idxc_v0, idxc_v1,
                  sem_in0, sem_in1, sem_out0, sem_out1):
    wid = lax.axis_index("s") * 2 + lax.axis_index("c")

    nk = (NCHUNK + 31) // 32
    feat_v = (feat_v0, feat_v1)
    idxc_v = (idxc_v0, idxc_v1)
    sem_in = (sem_in0, sem_in1)
    sem_out = (sem_out0, sem_out1)

    def in_dma(k, b):
        base = _chunk_base(wid + k * 32)
        return (
            pltpu.make_async_copy(idx_hbm.at[pl.ds(base, CH)],
                                  idxc_v[b].at[0], sem_in[b]),
            pltpu.make_async_copy(feat_hbm.at[pl.ds(base, CH), :],
                                  feat_v[b], sem_in[b]),
        )

    def out_dma(b):
        return pltpu.make_async_copy(feat_v[b], inter_hbm.at[idxc_v[b].at[0]],
                                     sem_out[b])

    def valid(k):
        return (wid + k * 32) < NCHUNK

    @pl.when(valid(0))
    def _():
        for d in in_dma(0, 0):
            d.start()

    for k in range(nk):
        b = k % 2

        if k + 1 < nk:
            @pl.when(valid(k + 1))
            def _(k=k):
                if k - 1 >= 0:
                    @pl.when(valid(k - 1))
                    def _(k=k):
                        out_dma((k + 1) % 2).wait()
                for d in in_dma(k + 1, (k + 1) % 2):
                    d.start()

        @pl.when(valid(k))
        def _(k=k, b=b):
            for d in in_dma(k, b):
                d.wait()
            out_dma(b).start()

    for k in (nk - 2, nk - 1):
        @pl.when(valid(k))
        def _(k=k):
            out_dma(k % 2).wait()


@functools.partial(jax.jit)
def _scatter_call(feat, idx):
    mesh = plsc.VectorSubcoreMesh(core_axis_name="c", subcore_axis_name="s")
    f = pl.kernel(
        _scatter_body,
        out_type=jax.ShapeDtypeStruct((S, C), jnp.float32),
        mesh=mesh,
        scratch_types=[
            pltpu.VMEM((CH, C), jnp.float32),
            pltpu.VMEM((CH, C), jnp.float32),
            pltpu.VMEM((1, CH), jnp.int32),
            pltpu.VMEM((1, CH), jnp.int32),
            pltpu.SemaphoreType.DMA,
            pltpu.SemaphoreType.DMA,
            pltpu.SemaphoreType.DMA,
            pltpu.SemaphoreType.DMA,
        ],
        compiler_params=pltpu.CompilerParams(needs_layout_passes=False),
    )
    return f(feat, idx)


def _transpose_body(inter_ref, mask_ref, out_ref):
    x = inter_ref[...]
    xt = x.T
    m = mask_ref[0, 0, :]
    o = jnp.where((m != 0)[None, :], xt, 0.0)
    out_ref[...] = o.reshape(C, T // NX, NX)


@functools.partial(jax.jit)
def _transpose_call(inter, mask3d):
    grid = (S // T,)
    yb = T // NX
    return pl.pallas_call(
        _transpose_body,
        grid=grid,
        in_specs=[
            pl.BlockSpec((T, C), lambda g: (g, 0)),
            pl.BlockSpec((1, 1, T), lambda g: (g, 0, 0)),
        ],
        out_specs=pl.BlockSpec((C, yb, NX), lambda g: (0, g, 0)),
        out_shape=jax.ShapeDtypeStruct((C, NY, NX), jnp.float32),
    )(inter, mask3d)


def kernel(pillar_features, coords):
    idx, mask = _idx_mask_call(coords.reshape(-1))
    inter = _scatter_call(pillar_features, idx)
    mask3d = mask.reshape(S // T, 1, T)
    out3 = _transpose_call(inter, mask3d)
    return out3.reshape(1, C, NY, NX)


# speedup vs baseline: 3.6495x; 1.1680x over previous
import functools

import jax
import jax.numpy as jnp
from jax import lax
from jax.experimental import pallas as pl
from jax.experimental.pallas import tpu as pltpu
from jax.experimental.pallas import tpu_sc as plsc

NX = 512
NY = 512
C = 128
P = 60000
S = NX * NY

CH = 128
NCHUNK = (P + CH - 1) // CH
LAST_BASE = P - CH
T = 8192
L = 16

NW = 32
NK = (NCHUNK + NW - 1) // NW
SLICE = P // L // NK


def _chunk_base(j):
    return jnp.minimum(j * CH, LAST_BASE)


FULL = 3752
LAST = P - 15 * FULL


def _idx_body(coords_hbm, idx_hbm, coords_v, idxst_v, sem):
    w = lax.axis_index("s")

    lanes4 = lax.iota(jnp.int32, L) * 4
    start = w * FULL

    @pl.when(w < 15)
    def _():
        pltpu.sync_copy(coords_hbm.at[pl.ds(start * 4, FULL * 4)],
                        coords_v.at[pl.ds(0, FULL * 4)])

    @pl.when(w == 15)
    def _():
        pltpu.sync_copy(coords_hbm.at[pl.ds(start * 4, LAST * 4)],
                        coords_v.at[pl.ds(0, LAST * 4)])

    niter = jnp.where(w < 15, (FULL + L - 1) // L, (LAST + L - 1) // L)

    def dec_step(i, _):
        base4 = i * (L * 4)
        y = plsc.load_gather(coords_v, [lanes4 + base4 + 2])
        x = plsc.load_gather(coords_v, [lanes4 + base4 + 3])
        idxst_v[pl.ds(i * L, L)] = y * NX + x
        return 0

    lax.fori_loop(0, niter, dec_step, 0)

    @pl.when(w < 15)
    def _():
        pltpu.sync_copy(idxst_v.at[pl.ds(0, FULL)],
                        idx_hbm.at[pl.ds(start, FULL)])

    @pl.when(w == 15)
    def _():
        pltpu.sync_copy(idxst_v.at[pl.ds(0, LAST)],
                        idx_hbm.at[pl.ds(start, LAST)])


@functools.partial(jax.jit)
def _idx_call(coords):
    mesh = plsc.VectorSubcoreMesh(core_axis_name="c", subcore_axis_name="s",
                                  num_cores=1)
    f = pl.kernel(
        _idx_body,
        out_type=jax.ShapeDtypeStruct((P,), jnp.int32),
        mesh=mesh,
        scratch_types=[
            pltpu.VMEM((FULL * 4 + 2 * L * 4,), jnp.int32),
            pltpu.VMEM((FULL + L,), jnp.int32),
            pltpu.SemaphoreType.DMA,
        ],
        compiler_params=pltpu.CompilerParams(needs_layout_passes=False,
                                             use_tc_tiling_on_sc=False),
    )
    return f(coords)


WIN = S // NW


def _scatter_body(feat_hbm, idx_hbm, inter_hbm, mask_hbm,
                  feat_v0, feat_v1, idxc_v0, idxc_v1, idx_full_v, win_v,
                  sem_in0, sem_in1, sem_out0, sem_out1, sem_idx):
    wid = lax.axis_index("s") * 2 + lax.axis_index("c")
    lo = wid * WIN

    feat_v = (feat_v0, feat_v1)
    idxc_v = (idxc_v0, idxc_v1)
    sem_in = (sem_in0, sem_in1)
    sem_out = (sem_out0, sem_out1)

    def in_dma(k, b):
        base = _chunk_base(wid + k * NW)
        return (
            pltpu.make_async_copy(idx_hbm.at[pl.ds(base, CH)],
                                  idxc_v[b].at[0], sem_in[b]),
            pltpu.make_async_copy(feat_hbm.at[pl.ds(base, CH), :],
                                  feat_v[b], sem_in[b]),
        )

    def out_dma(b):
        return pltpu.make_async_copy(feat_v[b], inter_hbm.at[idxc_v[b].at[0]],
                                     sem_out[b])

    def valid(k):
        return (wid + k * NW) < NCHUNK

    idx_full_dma = pltpu.make_async_copy(idx_hbm, idx_full_v, sem_idx)
    idx_full_dma.start()

    @pl.when(valid(0))
    def _():
        for d in in_dma(0, 0):
            d.start()

    def zero_step(i, _):
        win_v[pl.ds(i * L, L)] = jnp.zeros((L,), jnp.int32)
        return 0
    lax.fori_loop(0, WIN // L, zero_step, 0)

    ones16 = jnp.full((L,), 1, jnp.int32)

    def scan_step(i, _):
        v = idx_full_v[pl.ds(i * L, L)]
        local = v - lo
        m = (local >= 0) & (local < WIN)
        plsc.store_scatter(win_v, [jnp.where(m, local, 0)], ones16, mask=m)
        return 0

    for k in range(NK):
        b = k % 2

        if k + 1 < NK:
            @pl.when(valid(k + 1))
            def _(k=k):
                if k - 1 >= 0:
                    @pl.when(valid(k - 1))
                    def _(k=k):
                        out_dma((k + 1) % 2).wait()
                for d in in_dma(k + 1, (k + 1) % 2):
                    d.start()

        if k == 0:
            idx_full_dma.wait()
        lax.fori_loop(k * SLICE, (k + 1) * SLICE, scan_step, 0)

        @pl.when(valid(k))
        def _(k=k, b=b):
            for d in in_dma(k, b):
                d.wait()
            out_dma(b).start()

    for k in (NK - 2, NK - 1):
        @pl.when(valid(k))
        def _(k=k):
            out_dma(k % 2).wait()

    pltpu.sync_copy(win_v, mask_hbm.at[pl.ds(lo, WIN)])


@functools.partial(jax.jit)
def _scatter_call(feat, idx):
    mesh = plsc.VectorSubcoreMesh(core_axis_name="c", subcore_axis_name="s")
    f = pl.kernel(
        _scatter_body,
        out_type=(jax.ShapeDtypeStruct((S, C), jnp.float32),
                  jax.ShapeDtypeStruct((S,), jnp.int32)),
        mesh=mesh,
        scratch_types=[
            pltpu.VMEM((CH, C), jnp.float32),
            pltpu.VMEM((CH, C), jnp.float32),
            pltpu.VMEM((1, CH), jnp.int32),
            pltpu.VMEM((1, CH), jnp.int32),
            pltpu.VMEM((P,), jnp.int32),
            pltpu.VMEM((WIN,), jnp.int32),
            pltpu.SemaphoreType.DMA,
            pltpu.SemaphoreType.DMA,
            pltpu.SemaphoreType.DMA,
            pltpu.SemaphoreType.DMA,
            pltpu.SemaphoreType.DMA,
        ],
        compiler_params=pltpu.CompilerParams(needs_layout_passes=False),
    )
    return f(feat, idx)


def _transpose_body(inter_ref, mask_ref, out_ref):
    x = inter_ref[...]
    xt = x.T
    m = mask_ref[0, 0, :]
    o = jnp.where((m != 0)[None, :], xt, 0.0)
    out_ref[...] = o.reshape(C, T // NX, NX)


@functools.partial(jax.jit)
def _transpose_call(inter, mask3d):
    grid = (S // T,)
    yb = T // NX
    return pl.pallas_call(
        _transpose_body,
        grid=grid,
        in_specs=[
            pl.BlockSpec((T, C), lambda g: (g, 0)),
            pl.BlockSpec((1, 1, T), lambda g: (g, 0, 0)),
        ],
        out_specs=pl.BlockSpec((C, yb, NX), lambda g: (0, g, 0)),
        out_shape=jax.ShapeDtypeStruct((C, NY, NX), jnp.float32),
    )(inter, mask3d)


def kernel(pillar_features, coords):
    idx = _idx_call(coords.reshape(-1))
    inter, mask = _scatter_call(pillar_features, idx)
    mask3d = mask.reshape(S // T, 1, T)
    out3 = _transpose_call(inter, mask3d)
    return out3.reshape(1, C, NY, NX)


# speedup vs baseline: 3.6870x; 1.0103x over previous
import functools

import jax
import jax.numpy as jnp
from jax import lax
from jax.experimental import pallas as pl
from jax.experimental.pallas import tpu as pltpu
from jax.experimental.pallas import tpu_sc as plsc

NX = 512
NY = 512
C = 128
P = 60000
S = NX * NY

CH = 128
NCHUNK = (P + CH - 1) // CH
LAST_BASE = P - CH
T = 8192
L = 16

NW = 32
NK = (NCHUNK + NW - 1) // NW
SLICE = P // L // NK


def _chunk_base(j):
    return jnp.minimum(j * CH, LAST_BASE)


FULL = 3752
LAST = P - 15 * FULL


def _idx_body(coords_hbm, idx_hbm, coords_v, idxst_v, sem):
    w = lax.axis_index("s")

    lanes4 = lax.iota(jnp.int32, L) * 4
    start = w * FULL

    @pl.when(w < 15)
    def _():
        pltpu.sync_copy(coords_hbm.at[pl.ds(start * 4, FULL * 4)],
                        coords_v.at[pl.ds(0, FULL * 4)])

    @pl.when(w == 15)
    def _():
        pltpu.sync_copy(coords_hbm.at[pl.ds(start * 4, LAST * 4)],
                        coords_v.at[pl.ds(0, LAST * 4)])

    niter = jnp.where(w < 15, (FULL + L - 1) // L, (LAST + L - 1) // L)

    def dec_step(i, _):
        base4 = i * (L * 4)
        y = plsc.load_gather(coords_v, [lanes4 + base4 + 2])
        x = plsc.load_gather(coords_v, [lanes4 + base4 + 3])
        idxst_v[pl.ds(i * L, L)] = y * NX + x
        return 0

    lax.fori_loop(0, niter, dec_step, 0)

    @pl.when(w < 15)
    def _():
        pltpu.sync_copy(idxst_v.at[pl.ds(0, FULL)],
                        idx_hbm.at[pl.ds(start, FULL)])

    @pl.when(w == 15)
    def _():
        pltpu.sync_copy(idxst_v.at[pl.ds(0, LAST)],
                        idx_hbm.at[pl.ds(start, LAST)])


@functools.partial(jax.jit)
def _idx_call(coords):
    mesh = plsc.VectorSubcoreMesh(core_axis_name="c", subcore_axis_name="s",
                                  num_cores=1)
    f = pl.kernel(
        _idx_body,
        out_type=jax.ShapeDtypeStruct((P,), jnp.int32),
        mesh=mesh,
        scratch_types=[
            pltpu.VMEM((FULL * 4 + 2 * L * 4,), jnp.int32),
            pltpu.VMEM((FULL + L,), jnp.int32),
            pltpu.SemaphoreType.DMA,
        ],
        compiler_params=pltpu.CompilerParams(needs_layout_passes=False,
                                             use_tc_tiling_on_sc=False),
    )
    return f(coords)


WIN = S // NW


def _scatter_body(feat_hbm, idx_hbm, inter_hbm, mask_hbm,
                  feat_v0, feat_v1, idxc_v0, idxc_v1, idx_full_v, win_v,
                  sem_in0, sem_in1, sem_out0, sem_out1, sem_idx):
    wid = lax.axis_index("s") * 2 + lax.axis_index("c")
    lo = wid * WIN

    feat_v = (feat_v0, feat_v1)
    idxc_v = (idxc_v0, idxc_v1)
    sem_in = (sem_in0, sem_in1)
    sem_out = (sem_out0, sem_out1)

    def in_dma(k, b):
        base = _chunk_base(wid + k * NW)
        return (
            pltpu.make_async_copy(idx_hbm.at[pl.ds(base, CH)],
                                  idxc_v[b].at[0], sem_in[b]),
            pltpu.make_async_copy(feat_hbm.at[pl.ds(base, CH), :],
                                  feat_v[b], sem_in[b]),
        )

    def out_dma(b):
        return pltpu.make_async_copy(feat_v[b], inter_hbm.at[idxc_v[b].at[0]],
                                     sem_out[b])

    def valid(k):
        return (wid + k * NW) < NCHUNK

    idx_full_dma = pltpu.make_async_copy(idx_hbm, idx_full_v, sem_idx)
    idx_full_dma.start()

    @pl.when(valid(0))
    def _():
        for d in in_dma(0, 0):
            d.start()

    def zero_step(i, _):
        win_v[pl.ds(i * L, L)] = jnp.zeros((L,), jnp.int32)
        return 0
    lax.fori_loop(0, WIN // L, zero_step, 0)

    ones16 = jnp.full((L,), 1, jnp.int32)

    def scan_group(i):
        v = idx_full_v[pl.ds(i * L, L)]
        local = v - lo
        m = plsc.bitcast(local, jnp.uint32) < jnp.uint32(WIN)
        plsc.store_scatter(win_v, [jnp.where(m, local, 0)], ones16, mask=m)

    UNROLL = 5

    def scan_step(i, _):
        for u in range(UNROLL):
            scan_group(i * UNROLL + u)
        return 0

    for k in range(NK):
        b = k % 2

        if k + 1 < NK:
            @pl.when(valid(k + 1))
            def _(k=k):
                if k - 1 >= 0:
                    @pl.when(valid(k - 1))
                    def _(k=k):
                        out_dma((k + 1) % 2).wait()
                for d in in_dma(k + 1, (k + 1) % 2):
                    d.start()

        if k == 0:
            idx_full_dma.wait()
        lax.fori_loop(k * (SLICE // UNROLL), (k + 1) * (SLICE // UNROLL),
                      scan_step, 0)

        @pl.when(valid(k))
        def _(k=k, b=b):
            for d in in_dma(k, b):
                d.wait()
            out_dma(b).start()

    for k in (NK - 2, NK - 1):
        @pl.when(valid(k))
        def _(k=k):
            out_dma(k % 2).wait()

    pltpu.sync_copy(win_v, mask_hbm.at[pl.ds(lo, WIN)])


@functools.partial(jax.jit)
def _scatter_call(feat, idx):
    mesh = plsc.VectorSubcoreMesh(core_axis_name="c", subcore_axis_name="s")
    f = pl.kernel(
        _scatter_body,
        out_type=(jax.ShapeDtypeStruct((S, C), jnp.float32),
                  jax.ShapeDtypeStruct((S,), jnp.int32)),
        mesh=mesh,
        scratch_types=[
            pltpu.VMEM((CH, C), jnp.float32),
            pltpu.VMEM((CH, C), jnp.float32),
            pltpu.VMEM((1, CH), jnp.int32),
            pltpu.VMEM((1, CH), jnp.int32),
            pltpu.VMEM((P,), jnp.int32),
            pltpu.VMEM((WIN,), jnp.int32),
            pltpu.SemaphoreType.DMA,
            pltpu.SemaphoreType.DMA,
            pltpu.SemaphoreType.DMA,
            pltpu.SemaphoreType.DMA,
            pltpu.SemaphoreType.DMA,
        ],
        compiler_params=pltpu.CompilerParams(needs_layout_passes=False),
    )
    return f(feat, idx)


def _transpose_body(inter_ref, mask_ref, out_ref):
    x = inter_ref[...]
    xt = x.T
    m = mask_ref[0, 0, :]
    o = jnp.where((m != 0)[None, :], xt, 0.0)
    out_ref[...] = o.reshape(C, T // NX, NX)


@functools.partial(jax.jit)
def _transpose_call(inter, mask3d):
    grid = (S // T,)
    yb = T // NX
    return pl.pallas_call(
        _transpose_body,
        grid=grid,
        in_specs=[
            pl.BlockSpec((T, C), lambda g: (g, 0)),
            pl.BlockSpec((1, 1, T), lambda g: (g, 0, 0)),
        ],
        out_specs=pl.BlockSpec((C, yb, NX), lambda g: (0, g, 0)),
        out_shape=jax.ShapeDtypeStruct((C, NY, NX), jnp.float32),
    )(inter, mask3d)


def kernel(pillar_features, coords):
    idx = _idx_call(coords.reshape(-1))
    inter, mask = _scatter_call(pillar_features, idx)
    mask3d = mask.reshape(S // T, 1, T)
    out3 = _transpose_call(inter, mask3d)
    return out3.reshape(1, C, NY, NX)
